# baseline (speedup 1.0000x reference)
import jax
import jax.numpy as jnp
from jax import lax
from jax.experimental import pallas as pl
from jax.experimental.pallas import tpu as pltpu
from jax.experimental.pallas import tpu_sc as plsc

N_CAT = 26
N_NUM = 13
VOCAB = 100000
D = 32
T = N_CAT + N_NUM
TP1 = T + 1
W2 = 2 * D

NC = 2
NS = 16
NW = NC * NS

CAT_PW = N_CAT * D // NW
NUM_PW = N_NUM * D // NW
N_STATIC = T * D + W2
STA_PW = N_STATIC // NW

SUB = 4096
SSUB = 2048
NSUB = 4


def _num_mlp_body(xt_ref, w1_ref, b1_ref, w2_ref, b2_ref, out_ref):
    for n in range(N_NUM):
        h1t = (w1_ref[n][:, None] * xt_ref[n][None, :]
               + b1_ref[n][:, None])
        z = jax.lax.dot_general(
            w2_ref[n], h1t,
            dimension_numbers=(((1,), (0,)), ((), ())),
            preferred_element_type=jnp.float32,
        ) + b2_ref[n][:, None]
        out_ref[n] = jnp.tanh(z)


def _num_mlp(xt, num_w1, num_b1, num_w2, num_b2):
    B = xt.shape[1]
    BB = 2048
    grid = (B // BB,)
    return pl.pallas_call(
        _num_mlp_body,
        grid=grid,
        in_specs=[
            pl.BlockSpec((N_NUM, BB), lambda i: (0, i)),
            pl.BlockSpec((N_NUM, D), lambda i: (0, 0)),
            pl.BlockSpec((N_NUM, D), lambda i: (0, 0)),
            pl.BlockSpec((N_NUM, D, D), lambda i: (0, 0, 0)),
            pl.BlockSpec((N_NUM, D), lambda i: (0, 0)),
        ],
        out_specs=pl.BlockSpec((N_NUM, D, BB), lambda i: (0, 0, i)),
        out_shape=jax.ShapeDtypeStruct((N_NUM, D, B), jnp.float32),
    )(xt, num_w1, num_b1, num_w2, num_b2)


HALF = 50048
HB0 = 49920
HB_LEN = VOCAB - HB0


def _sc_body(table_ref, xcat_ref, num_ref, const_ref, out_ref,
             col_a, col_b, res0, res1, res2, idx0, idx1, sbuf, nbuf, cvm,
             casem, cbsem, wsem, ssem, nsem, isem):
    res_l = [res0, res1, res2]
    idx_l = [idx0, idx1]
    w = lax.axis_index("s") * NC + lax.axis_index("c")

    pltpu.sync_copy(const_ref, cvm)

    def out_at(t, c, h):
        return out_ref.at[t, c, pl.ds(h * SUB, SUB)]

    def drain(n, buf, sem):
        for _ in range(n):
            pltpu.make_async_copy(buf, out_at(0, 0, 0), sem).wait()

    def static_plane(s, first):
        @pl.when(jnp.logical_not(first))
        def _():
            for _ in range(2 * NSUB):
                pltpu.make_async_copy(
                    sbuf, out_ref.at[0, 0, pl.ds(0, SSUB)], ssem).wait()

        is_cls = s >= T * D
        t = jnp.where(is_cls, T, s // D)
        c = jnp.where(is_cls, s - T * D, D + s % D)
        iv = jnp.full((16,), t * W2 + c, jnp.int32)
        vec = plsc.load_gather(cvm, [iv])

        @pl.loop(0, SSUB // 16)
        def _(i):
            sbuf[pl.ds(i * 16, 16)] = vec

        for h in range(2 * NSUB):
            pltpu.async_copy(
                sbuf, out_ref.at[t, c, pl.ds(h * SSUB, SSUB)], ssem)

    def num_plane(q, first):
        j = q // D
        c = q % D
        for h in range(NSUB):
            if h > 0:
                drain(1, nbuf, nsem)
            else:
                @pl.when(jnp.logical_not(first))
                def _():
                    drain(1, nbuf, nsem)
            pltpu.sync_copy(num_ref.at[j, c, pl.ds(h * SUB, SUB)], nbuf)
            pltpu.async_copy(nbuf, out_at(N_CAT + j, c, h), nsem)

    pltpu.async_copy(table_ref.at[pl.ds(w * CAT_PW, 1), pl.ds(0, HALF)],
                     col_a, casem)

    @pl.loop(0, CAT_PW)
    def _cat(k):
        p = w * CAT_PW + k
        f = p // D
        c = p % D

        cb = pltpu.async_copy(table_ref.at[pl.ds(p, 1), pl.ds(HB0, HB_LEN)],
                              col_b, cbsem)

        @pl.when(k < NUM_PW)
        def _():
            num_plane(w * NUM_PW + k, k == 0)

        static_plane(w * STA_PW + k, k == 0)

        @pl.when(k < STA_PW - CAT_PW)
        def _():
            static_plane(w * STA_PW + CAT_PW + k, False)

        pltpu.make_async_copy(
            table_ref.at[pl.ds(p, 1), pl.ds(0, HALF)], col_a, casem).wait()

        def gather_lo(ii, slot):
            idx_r, res_r = idx_l[ii], res_l[slot]

            @pl.loop(0, SUB // 64)
            def _(i):
                for u in range(4):
                    o = i * 64 + u * 16
                    iv = idx_r[pl.ds(o, 16)]
                    iv_a = jnp.minimum(iv, HALF - 1)
                    iv0 = jnp.zeros((16,), jnp.int32)
                    res_r[pl.ds(o, 16)] = plsc.load_gather(col_a,
                                                           [iv0, iv_a])

        def gather_hi(ii, slot):
            idx_r, res_r = idx_l[ii], res_l[slot]

            @pl.loop(0, SUB // 64)
            def _(i):
                for u in range(4):
                    o = i * 64 + u * 16
                    iv = idx_r[pl.ds(o, 16)]
                    m = iv >= HB0
                    iv_b = jnp.maximum(iv - HB0, 0)
                    iv0 = jnp.zeros((16,), jnp.int32)
                    g_b = plsc.load_gather(col_b, [iv0, iv_b])
                    cur = res_r[pl.ds(o, 16)]
                    res_r[pl.ds(o, 16)] = jnp.where(m, g_b, cur)

        for g2 in range(2):
            h0, h1 = 2 * g2, 2 * g2 + 1
            s0, s1 = (0, 1) if g2 == 0 else (2, 0)
            di0 = pltpu.async_copy(xcat_ref.at[f, pl.ds(h0 * SUB, SUB)],
                                   idx0, isem)
            di1 = pltpu.async_copy(xcat_ref.at[f, pl.ds(h1 * SUB, SUB)],
                                   idx1, isem)

            if g2 == 0:
                @pl.when(k > 0)
                def _():
                    drain(2, res0, wsem)
            else:
                drain(2, res0, wsem)

            di0.wait()
            di1.wait()
            gather_lo(0, s0)
            gather_lo(1, s1)

            if g2 == 0:
                cb.wait()
            else:
                @pl.when(k + 1 < CAT_PW)
                def _():
                    pltpu.async_copy(
                        table_ref.at[pl.ds(p + 1, 1), pl.ds(0, HALF)],
                        col_a, casem)

            gather_hi(0, s0)
            gather_hi(1, s1)

            pltpu.async_copy(res_l[s0], out_at(f, c, h0), wsem)
            pltpu.async_copy(res_l[s1], out_at(f, c, h1), wsem)

    drain(2, res0, wsem)
    for _ in range(2 * NSUB):
        pltpu.make_async_copy(sbuf, out_ref.at[0, 0, pl.ds(0, SSUB)],
                              ssem).wait()
    drain(1, nbuf, nsem)


def _sc_assemble(table2, xcat_t, num_planes, consts):
    B = xcat_t.shape[1]
    mesh = plsc.VectorSubcoreMesh(core_axis_name="c", subcore_axis_name="s")
    kern = pl.kernel(
        _sc_body,
        out_type=jax.ShapeDtypeStruct((TP1, W2, B), jnp.float32),
        mesh=mesh,
        scratch_types=[
            pltpu.VMEM((1, HALF), jnp.float32),
            pltpu.VMEM((1, HB_LEN), jnp.float32),
            pltpu.VMEM((SUB,), jnp.float32),
            pltpu.VMEM((SUB,), jnp.float32),
            pltpu.VMEM((SUB,), jnp.float32),
            pltpu.VMEM((SUB,), jnp.int32),
            pltpu.VMEM((SUB,), jnp.int32),
            pltpu.VMEM((SSUB,), jnp.float32),
            pltpu.VMEM((SUB,), jnp.float32),
            pltpu.VMEM((TP1 * W2,), jnp.float32),
            pltpu.SemaphoreType.DMA,
            pltpu.SemaphoreType.DMA,
            pltpu.SemaphoreType.DMA,
            pltpu.SemaphoreType.DMA,
            pltpu.SemaphoreType.DMA,
            pltpu.SemaphoreType.DMA,
        ],
        compiler_params=pltpu.CompilerParams(use_tc_tiling_on_sc=True,
                                             needs_layout_passes=False),
    )
    return kern(table2, xcat_t, num_planes, consts)


def kernel(x_cat, x_num, cat_tables, num_w1, num_b1, num_w2, num_b2,
           pos_table, cls_token):
    B = x_cat.shape[0]

    table2 = cat_tables.transpose(0, 2, 1).reshape(N_CAT * D, VOCAB)
    xcat_t = x_cat.T
    xt = x_num.T

    cls = cls_token.reshape(W2)
    consts = jnp.zeros((TP1, W2), jnp.float32)
    consts = consts.at[:T, D:].set(pos_table)
    consts = consts.at[T, :].set(cls)
    consts = consts.reshape(TP1 * W2)

    num_planes = _num_mlp(xt, num_w1, num_b1, num_w2, num_b2)
    out_phys = _sc_assemble(table2, xcat_t, num_planes, consts)
    return out_phys.transpose(2, 0, 1)
